# baseline (speedup 1.0000x reference)
import functools

import jax
import jax.numpy as jnp
from jax import lax
from jax.experimental import pallas as pl
from jax.experimental.pallas import tpu as pltpu
from jax.experimental.pallas import tpu_sc as plsc

B = 4
N = 4096
C = 256
TILE = 256
NT = N // TILE
BN = B * N


def _top2_body(q_ref, pt_ref, i0_ref, i1_ref):
    b = pl.program_id(0)
    q = q_ref[0]
    pt = pt_ref[0]
    dot = jnp.dot(q, pt, preferred_element_type=jnp.float32)
    q2 = jnp.sum(q * q, axis=1, keepdims=True)
    p2 = jnp.sum(pt * pt, axis=0, keepdims=True)
    d2 = jnp.maximum(q2 + p2 - 2.0 * dot, 0.0)
    iota = lax.broadcasted_iota(jnp.int32, (TILE, N), 1)
    inf = jnp.float32(jnp.inf)
    m1 = jnp.min(d2, axis=1, keepdims=True)
    idx1 = jnp.min(jnp.where(d2 == m1, iota, N), axis=1, keepdims=True)
    d2x = jnp.where(iota == idx1, inf, d2)
    m2 = jnp.min(d2x, axis=1, keepdims=True)
    idx2 = jnp.min(jnp.where(d2x == m2, iota, N), axis=1, keepdims=True)
    off = b * N
    i0_ref[...] = idx1 + off
    i1_ref[...] = idx2 + off


def _top2_indices(pts_pad, pts_t):
    idx_shape = jax.ShapeDtypeStruct((BN, 1), jnp.int32)
    return pl.pallas_call(
        _top2_body,
        grid=(B, NT),
        in_specs=[
            pl.BlockSpec((1, TILE, 8), lambda b, t: (b, t, 0)),
            pl.BlockSpec((1, 8, N), lambda b, t: (b, 0, 0)),
        ],
        out_specs=[
            pl.BlockSpec((TILE, 1), lambda b, t: (b * NT + t, 0)),
            pl.BlockSpec((TILE, 1), lambda b, t: (b * NT + t, 0)),
        ],
        out_shape=[idx_shape, idx_shape],
    )(pts_pad, pts_t)


def _gather_mean(preds_flat, i0, i1):
    info = plsc.get_sparse_core_info()
    nc, ns = info.num_cores, info.num_subcores
    nw = nc * ns
    per_w = BN // nw
    ch = 128
    n_ch = per_w // ch

    mesh = plsc.VectorSubcoreMesh(core_axis_name="c", subcore_axis_name="s")

    @functools.partial(
        pl.kernel,
        mesh=mesh,
        out_type=jax.ShapeDtypeStruct((BN, C), jnp.float32),
        scratch_types=[
            pltpu.VMEM((ch,), jnp.int32),
            pltpu.VMEM((ch,), jnp.int32),
            pltpu.VMEM((ch, C), jnp.float32),
            pltpu.VMEM((ch, C), jnp.float32),
            pltpu.SemaphoreType.DMA,
            pltpu.SemaphoreType.DMA,
        ],
    )
    def body(preds_hbm, i0_hbm, i1_hbm, out_hbm, i0_v, i1_v, r0_v, r1_v, s0, s1):
        wid = lax.axis_index("s") * nc + lax.axis_index("c")
        base = wid * per_w
        for c in range(n_ch):
            off = base + c * ch
            pltpu.sync_copy(i0_hbm.at[pl.ds(off, ch)], i0_v)
            pltpu.sync_copy(i1_hbm.at[pl.ds(off, ch)], i1_v)
            cp0 = pltpu.async_copy(preds_hbm.at[i0_v], r0_v, s0)
            cp1 = pltpu.async_copy(preds_hbm.at[i1_v], r1_v, s1)
            cp0.wait()
            cp1.wait()

            def row_body(r, carry):
                for j in range(C // 16):
                    sl = pl.ds(j * 16, 16)
                    r0_v[r, sl] = (r0_v[r, sl] + r1_v[r, sl]) * 0.5
                return carry

            lax.fori_loop(0, ch, row_body, 0)
            pltpu.sync_copy(r0_v, out_hbm.at[pl.ds(off, ch)])

    return body(preds_flat, i0, i1)


def kernel(points, preds, k_vector):
    del k_vector
    pts_pad = jnp.pad(points, ((0, 0), (0, 0), (0, 5)))
    pts_t = jnp.transpose(pts_pad, (0, 2, 1))
    i0, i1 = _top2_indices(pts_pad, pts_t)
    out = _gather_mean(preds.reshape(BN, C), i0.reshape(BN), i1.reshape(BN))
    return out.reshape(B, N, C)


# speedup vs baseline: 1.2317x; 1.2317x over previous
import functools

import jax
import jax.numpy as jnp
from jax import lax
from jax.experimental import pallas as pl
from jax.experimental.pallas import tpu as pltpu
from jax.experimental.pallas import tpu_sc as plsc

B = 4
N = 4096
C = 256
TILE = 512
NT = N // TILE
BN = B * N


def _top2_body(q_ref, pt_ref, i0_ref, i1_ref):
    b = pl.program_id(0)
    q = q_ref[0]
    pt = pt_ref[0]
    dot = jnp.dot(q, pt, preferred_element_type=jnp.float32)
    q2 = jnp.sum(q * q, axis=1, keepdims=True)
    p2 = jnp.sum(pt * pt, axis=0, keepdims=True)
    d2 = jnp.maximum(q2 + p2 - 2.0 * dot, 0.0)
    fiota = lax.broadcasted_iota(jnp.int32, (TILE, N), 1).astype(jnp.float32)
    big = jnp.float32(N)
    inf = jnp.float32(jnp.inf)
    m1 = jnp.min(d2, axis=1, keepdims=True)
    fidx1 = jnp.min(jnp.where(d2 == m1, fiota, big), axis=1, keepdims=True)
    d2x = jnp.where(fiota == fidx1, inf, d2)
    m2 = jnp.min(d2x, axis=1, keepdims=True)
    fidx2 = jnp.min(jnp.where(d2x == m2, fiota, big), axis=1, keepdims=True)
    off = b * N
    i0_ref[...] = fidx1.astype(jnp.int32) + off
    i1_ref[...] = fidx2.astype(jnp.int32) + off


def _top2_indices(pts_pad, pts_t):
    idx_shape = jax.ShapeDtypeStruct((BN, 1), jnp.int32)
    return pl.pallas_call(
        _top2_body,
        grid=(B, NT),
        in_specs=[
            pl.BlockSpec((1, TILE, 8), lambda b, t: (b, t, 0)),
            pl.BlockSpec((1, 8, N), lambda b, t: (b, 0, 0)),
        ],
        out_specs=[
            pl.BlockSpec((TILE, 1), lambda b, t: (b * NT + t, 0)),
            pl.BlockSpec((TILE, 1), lambda b, t: (b * NT + t, 0)),
        ],
        out_shape=[idx_shape, idx_shape],
    )(pts_pad, pts_t)


def _gather_mean(preds_flat, i0, i1):
    info = plsc.get_sparse_core_info()
    nc, ns = info.num_cores, info.num_subcores
    nw = nc * ns
    per_w = BN // nw
    ch = 64
    n_ch = per_w // ch

    mesh = plsc.VectorSubcoreMesh(core_axis_name="c", subcore_axis_name="s")

    @functools.partial(
        pl.kernel,
        mesh=mesh,
        out_type=jax.ShapeDtypeStruct((BN, C), jnp.float32),
        scratch_types=[
            pltpu.VMEM((per_w,), jnp.int32),
            pltpu.VMEM((per_w,), jnp.int32),
            pltpu.VMEM((ch, C), jnp.float32),
            pltpu.VMEM((ch, C), jnp.float32),
            pltpu.VMEM((ch, C), jnp.float32),
            pltpu.VMEM((ch, C), jnp.float32),
            pltpu.SemaphoreType.DMA,
            pltpu.SemaphoreType.DMA,
            pltpu.SemaphoreType.DMA,
            pltpu.SemaphoreType.DMA,
        ],
    )
    def body(preds_hbm, i0_hbm, i1_hbm, out_hbm,
             i0_v, i1_v, r0a, r1a, r0b, r1b, sga, sgb, soa, sob):
        wid = lax.axis_index("s") * nc + lax.axis_index("c")
        base = wid * per_w
        pltpu.sync_copy(i0_hbm.at[pl.ds(base, per_w)], i0_v)
        pltpu.sync_copy(i1_hbm.at[pl.ds(base, per_w)], i1_v)
        bufs = [(r0a, r1a, sga, soa), (r0b, r1b, sgb, sob)]
        gat = [None, None]
        out = [None, None]

        def issue_gather(c):
            r0, r1, sg, _ = bufs[c % 2]
            sl = pl.ds(c * ch, ch)
            a = pltpu.async_copy(preds_hbm.at[i0_v.at[sl]], r0, sg)
            b = pltpu.async_copy(preds_hbm.at[i1_v.at[sl]], r1, sg)
            gat[c % 2] = (a, b)

        issue_gather(0)
        for c in range(n_ch):
            s = c % 2
            r0, r1, _, so = bufs[s]
            ga, gb = gat[s]
            ga.wait()
            gb.wait()
            if c + 1 < n_ch:
                if out[(c + 1) % 2] is not None:
                    out[(c + 1) % 2].wait()
                issue_gather(c + 1)

            def row_body(r, carry, r0=r0, r1=r1):
                for j in range(C // 16):
                    sl = pl.ds(j * 16, 16)
                    r0[r, sl] = (r0[r, sl] + r1[r, sl]) * 0.5
                return carry

            lax.fori_loop(0, ch, row_body, 0)
            out[s] = pltpu.async_copy(r0, out_hbm.at[pl.ds(base + c * ch, ch)], so)
        out[(n_ch - 1) % 2].wait()
        out[n_ch % 2].wait()

    return body(preds_flat, i0, i1)


def kernel(points, preds, k_vector):
    del k_vector
    pts_pad = jnp.pad(points, ((0, 0), (0, 0), (0, 5)))
    pts_t = jnp.transpose(pts_pad, (0, 2, 1))
    i0, i1 = _top2_indices(pts_pad, pts_t)
    out = _gather_mean(preds.reshape(BN, C), i0.reshape(BN), i1.reshape(BN))
    return out.reshape(B, N, C)


# speedup vs baseline: 1.2605x; 1.0235x over previous
import functools

import jax
import jax.numpy as jnp
from jax import lax
from jax.experimental import pallas as pl
from jax.experimental.pallas import tpu as pltpu
from jax.experimental.pallas import tpu_sc as plsc

B = 4
N = 4096
C = 256
TILE = 1024
NT = N // TILE
BN = B * N


def _top2_body(p_ref, q_ref, i0_ref, i1_ref):
    b = pl.program_id(0)
    q = p_ref[0]
    pt = q_ref[0]

    dot = jnp.dot(q, pt, preferred_element_type=jnp.float32)
    q2 = jnp.sum(q * q, axis=1, keepdims=True)
    p2 = jnp.sum(pt * pt, axis=0, keepdims=True)
    d2 = jnp.maximum(q2 + p2 - 2.0 * dot, 0.0)
    fiota = lax.broadcasted_iota(jnp.int32, (TILE, N), 1).astype(jnp.float32)
    big = jnp.float32(N)
    inf = jnp.float32(jnp.inf)
    m1 = jnp.min(d2, axis=1, keepdims=True)
    fidx1 = jnp.min(jnp.where(d2 == m1, fiota, big), axis=1, keepdims=True)
    d2x = jnp.where(fiota == fidx1, inf, d2)
    m2 = jnp.min(d2x, axis=1, keepdims=True)
    fidx2 = jnp.min(jnp.where(d2x == m2, fiota, big), axis=1, keepdims=True)
    off = b * N
    i0_ref[...] = fidx1.astype(jnp.int32) + off
    i1_ref[...] = fidx2.astype(jnp.int32) + off


def _top2_indices(pts_pad, pts_t):
    idx_shape = jax.ShapeDtypeStruct((BN, 1), jnp.int32)
    return pl.pallas_call(
        _top2_body,
        grid=(B, NT),
        in_specs=[
            pl.BlockSpec((1, TILE, 8), lambda b, t: (b, t, 0)),
            pl.BlockSpec((1, 8, N), lambda b, t: (b, 0, 0)),
        ],
        out_specs=[
            pl.BlockSpec((TILE, 1), lambda b, t: (b * NT + t, 0)),
            pl.BlockSpec((TILE, 1), lambda b, t: (b * NT + t, 0)),
        ],
        out_shape=[idx_shape, idx_shape],
    )(pts_pad, pts_t)


def _gather_mean(preds_flat, i0, i1):
    info = plsc.get_sparse_core_info()
    nc, ns = info.num_cores, info.num_subcores
    nw = nc * ns
    per_w = BN // nw
    ch = 64
    n_ch = per_w // ch

    mesh = plsc.VectorSubcoreMesh(core_axis_name="c", subcore_axis_name="s")

    @functools.partial(
        pl.kernel,
        mesh=mesh,
        out_type=jax.ShapeDtypeStruct((BN, C), jnp.float32),
        scratch_types=[
            pltpu.VMEM((per_w,), jnp.int32),
            pltpu.VMEM((per_w,), jnp.int32),
            pltpu.VMEM((ch, C), jnp.float32),
            pltpu.VMEM((ch, C), jnp.float32),
            pltpu.VMEM((ch, C), jnp.float32),
            pltpu.VMEM((ch, C), jnp.float32),
            pltpu.SemaphoreType.DMA,
            pltpu.SemaphoreType.DMA,
            pltpu.SemaphoreType.DMA,
            pltpu.SemaphoreType.DMA,
        ],
    )
    def body(preds_hbm, i0_hbm, i1_hbm, out_hbm,
             i0_v, i1_v, r0a, r1a, r0b, r1b, sga, sgb, soa, sob):
        wid = lax.axis_index("s") * nc + lax.axis_index("c")
        base = wid * per_w
        pltpu.sync_copy(i0_hbm.at[pl.ds(base, per_w)], i0_v)
        pltpu.sync_copy(i1_hbm.at[pl.ds(base, per_w)], i1_v)
        bufs = [(r0a, r1a, sga, soa), (r0b, r1b, sgb, sob)]
        gat = [None, None]
        out = [None, None]

        def issue_gather(c):
            r0, r1, sg, _ = bufs[c % 2]
            sl = pl.ds(c * ch, ch)
            a = pltpu.async_copy(preds_hbm.at[i0_v.at[sl]], r0, sg)
            b = pltpu.async_copy(preds_hbm.at[i1_v.at[sl]], r1, sg)
            gat[c % 2] = (a, b)

        issue_gather(0)
        for c in range(n_ch):
            s = c % 2
            r0, r1, _, so = bufs[s]
            ga, gb = gat[s]
            ga.wait()
            gb.wait()
            if c + 1 < n_ch:
                if out[(c + 1) % 2] is not None:
                    out[(c + 1) % 2].wait()
                issue_gather(c + 1)

            def row_body(r, carry, r0=r0, r1=r1):
                for j in range(C // 16):
                    sl = pl.ds(j * 16, 16)
                    r0[r, sl] = (r0[r, sl] + r1[r, sl]) * 0.5
                return carry

            lax.fori_loop(0, ch, row_body, 0)
            out[s] = pltpu.async_copy(r0, out_hbm.at[pl.ds(base + c * ch, ch)], so)
        out[(n_ch - 1) % 2].wait()
        out[n_ch % 2].wait()

    return body(preds_flat, i0, i1)


def kernel(points, preds, k_vector):
    del k_vector
    pts_pad = jnp.pad(points, ((0, 0), (0, 0), (0, 5)))
    pts_t = jnp.transpose(pts_pad, (0, 2, 1))
    i0, i1 = _top2_indices(pts_pad, pts_t)
    out = _gather_mean(preds.reshape(BN, C), i0.reshape(BN), i1.reshape(BN))
    return out.reshape(B, N, C)


# speedup vs baseline: 1.3046x; 1.0350x over previous
import functools

import jax
import jax.numpy as jnp
from jax import lax
from jax.experimental import pallas as pl
from jax.experimental.pallas import tpu as pltpu
from jax.experimental.pallas import tpu_sc as plsc

B = 4
N = 4096
C = 256
TILE = 1024
NT = N // TILE
BN = B * N


def _top2_body(p_ref, q_ref, i0_ref, i1_ref):
    b = pl.program_id(0)
    q = p_ref[0]
    pt = q_ref[0]

    dot = jnp.dot(q, pt, preferred_element_type=jnp.float32)
    q2 = jnp.sum(q * q, axis=1, keepdims=True)
    p2 = jnp.sum(pt * pt, axis=0, keepdims=True)
    d2 = jnp.maximum(q2 + p2 - 2.0 * dot, 0.0)
    fiota = lax.broadcasted_iota(jnp.int32, (TILE, N), 1).astype(jnp.float32)
    big = jnp.float32(N)
    inf = jnp.float32(jnp.inf)
    m1 = jnp.min(d2, axis=1, keepdims=True)
    fidx1 = jnp.min(jnp.where(d2 == m1, fiota, big), axis=1, keepdims=True)
    d2x = jnp.where(fiota == fidx1, inf, d2)
    m2 = jnp.min(d2x, axis=1, keepdims=True)
    fidx2 = jnp.min(jnp.where(d2x == m2, fiota, big), axis=1, keepdims=True)
    off = b * N
    i0_ref[...] = fidx1.astype(jnp.int32) + off
    i1_ref[...] = fidx2.astype(jnp.int32) + off


def _top2_indices(pts_pad, pts_t):
    idx_shape = jax.ShapeDtypeStruct((BN, 1), jnp.int32)
    return pl.pallas_call(
        _top2_body,
        grid=(B, NT),
        in_specs=[
            pl.BlockSpec((1, TILE, 3), lambda b, t: (b, t, 0)),
            pl.BlockSpec((1, 3, N), lambda b, t: (b, 0, 0)),
        ],
        out_specs=[
            pl.BlockSpec((TILE, 1), lambda b, t: (b * NT + t, 0)),
            pl.BlockSpec((TILE, 1), lambda b, t: (b * NT + t, 0)),
        ],
        out_shape=[idx_shape, idx_shape],
    )(pts_pad, pts_t)


def _gather_mean(preds_flat, i0, i1):
    info = plsc.get_sparse_core_info()
    nc, ns = info.num_cores, info.num_subcores
    nw = nc * ns
    per_w = BN // nw
    ch = 64
    n_ch = per_w // ch

    mesh = plsc.VectorSubcoreMesh(core_axis_name="c", subcore_axis_name="s")

    @functools.partial(
        pl.kernel,
        mesh=mesh,
        out_type=jax.ShapeDtypeStruct((BN, C), jnp.float32),
        scratch_types=[
            pltpu.VMEM((per_w,), jnp.int32),
            pltpu.VMEM((per_w,), jnp.int32),
            pltpu.VMEM((ch, C), jnp.float32),
            pltpu.VMEM((ch, C), jnp.float32),
            pltpu.VMEM((ch, C), jnp.float32),
            pltpu.VMEM((ch, C), jnp.float32),
            pltpu.SemaphoreType.DMA,
            pltpu.SemaphoreType.DMA,
            pltpu.SemaphoreType.DMA,
            pltpu.SemaphoreType.DMA,
        ],
    )
    def body(preds_hbm, i0_hbm, i1_hbm, out_hbm,
             i0_v, i1_v, r0a, r1a, r0b, r1b, sga, sgb, soa, sob):
        wid = lax.axis_index("s") * nc + lax.axis_index("c")
        base = wid * per_w
        pltpu.sync_copy(i0_hbm.at[pl.ds(base, per_w)], i0_v)
        pltpu.sync_copy(i1_hbm.at[pl.ds(base, per_w)], i1_v)
        bufs = [(r0a, r1a, sga, soa), (r0b, r1b, sgb, sob)]
        gat = [None, None]
        out = [None, None]

        def issue_gather(c):
            r0, r1, sg, _ = bufs[c % 2]
            sl = pl.ds(c * ch, ch)
            a = pltpu.async_copy(preds_hbm.at[i0_v.at[sl]], r0, sg)
            b = pltpu.async_copy(preds_hbm.at[i1_v.at[sl]], r1, sg)
            gat[c % 2] = (a, b)

        issue_gather(0)
        for c in range(n_ch):
            s = c % 2
            r0, r1, _, so = bufs[s]
            ga, gb = gat[s]
            ga.wait()
            gb.wait()
            if c + 1 < n_ch:
                if out[(c + 1) % 2] is not None:
                    out[(c + 1) % 2].wait()
                issue_gather(c + 1)

            def row_body(r, carry, r0=r0, r1=r1):
                for j in range(C // 16):
                    sl = pl.ds(j * 16, 16)
                    r0[r, sl] = (r0[r, sl] + r1[r, sl]) * 0.5
                return carry

            lax.fori_loop(0, ch, row_body, 0)
            out[s] = pltpu.async_copy(r0, out_hbm.at[pl.ds(base + c * ch, ch)], so)
        out[(n_ch - 1) % 2].wait()
        out[n_ch % 2].wait()

    return body(preds_flat, i0, i1)


def kernel(points, preds, k_vector):
    del k_vector
    pts_t = jnp.transpose(points, (0, 2, 1))
    i0, i1 = _top2_indices(points, pts_t)
    out = _gather_mean(preds.reshape(BN, C), i0.reshape(BN), i1.reshape(BN))
    return out.reshape(B, N, C)


# speedup vs baseline: 1.3130x; 1.0064x over previous
import functools

import jax
import jax.numpy as jnp
from jax import lax
from jax.experimental import pallas as pl
from jax.experimental.pallas import tpu as pltpu
from jax.experimental.pallas import tpu_sc as plsc

B = 4
N = 4096
C = 256
TILE = 2048
NT = N // TILE
BN = B * N


def _top2_body(p_ref, q_ref, i0_ref, i1_ref):
    b = pl.program_id(0)
    q = p_ref[0]
    pt = q_ref[0]

    dot = jnp.dot(q, pt, preferred_element_type=jnp.float32)
    q2 = jnp.sum(q * q, axis=1, keepdims=True)
    p2 = jnp.sum(pt * pt, axis=0, keepdims=True)
    d2 = jnp.maximum(q2 + p2 - 2.0 * dot, 0.0)
    fiota = lax.broadcasted_iota(jnp.int32, (TILE, N), 1).astype(jnp.float32)
    big = jnp.float32(N)
    inf = jnp.float32(jnp.inf)
    m1 = jnp.min(d2, axis=1, keepdims=True)
    fidx1 = jnp.min(jnp.where(d2 == m1, fiota, big), axis=1, keepdims=True)
    d2x = jnp.where(fiota == fidx1, inf, d2)
    m2 = jnp.min(d2x, axis=1, keepdims=True)
    fidx2 = jnp.min(jnp.where(d2x == m2, fiota, big), axis=1, keepdims=True)
    off = b * N
    i0_ref[...] = fidx1.astype(jnp.int32) + off
    i1_ref[...] = fidx2.astype(jnp.int32) + off


def _top2_indices(pts_pad, pts_t):
    idx_shape = jax.ShapeDtypeStruct((BN, 1), jnp.int32)
    return pl.pallas_call(
        _top2_body,
        grid=(B, NT),
        in_specs=[
            pl.BlockSpec((1, TILE, 3), lambda b, t: (b, t, 0)),
            pl.BlockSpec((1, 3, N), lambda b, t: (b, 0, 0)),
        ],
        out_specs=[
            pl.BlockSpec((TILE, 1), lambda b, t: (b * NT + t, 0)),
            pl.BlockSpec((TILE, 1), lambda b, t: (b * NT + t, 0)),
        ],
        out_shape=[idx_shape, idx_shape],
    )(pts_pad, pts_t)


def _gather_mean(preds_flat, i0, i1):
    info = plsc.get_sparse_core_info()
    nc, ns = info.num_cores, info.num_subcores
    nw = nc * ns
    per_w = BN // nw
    ch = 64
    n_ch = per_w // ch

    mesh = plsc.VectorSubcoreMesh(core_axis_name="c", subcore_axis_name="s")

    @functools.partial(
        pl.kernel,
        mesh=mesh,
        out_type=jax.ShapeDtypeStruct((BN, C), jnp.float32),
        scratch_types=[
            pltpu.VMEM((per_w,), jnp.int32),
            pltpu.VMEM((per_w,), jnp.int32),
            pltpu.VMEM((ch, C), jnp.float32),
            pltpu.VMEM((ch, C), jnp.float32),
            pltpu.VMEM((ch, C), jnp.float32),
            pltpu.VMEM((ch, C), jnp.float32),
            pltpu.SemaphoreType.DMA,
            pltpu.SemaphoreType.DMA,
            pltpu.SemaphoreType.DMA,
            pltpu.SemaphoreType.DMA,
        ],
    )
    def body(preds_hbm, i0_hbm, i1_hbm, out_hbm,
             i0_v, i1_v, r0a, r1a, r0b, r1b, sga, sgb, soa, sob):
        wid = lax.axis_index("s") * nc + lax.axis_index("c")
        base = wid * per_w
        pltpu.sync_copy(i0_hbm.at[pl.ds(base, per_w)], i0_v)
        pltpu.sync_copy(i1_hbm.at[pl.ds(base, per_w)], i1_v)
        bufs = [(r0a, r1a, sga, soa), (r0b, r1b, sgb, sob)]
        gat = [None, None]
        out = [None, None]

        def issue_gather(c):
            r0, r1, sg, _ = bufs[c % 2]
            sl = pl.ds(c * ch, ch)
            a = pltpu.async_copy(preds_hbm.at[i0_v.at[sl]], r0, sg)
            b = pltpu.async_copy(preds_hbm.at[i1_v.at[sl]], r1, sg)
            gat[c % 2] = (a, b)

        issue_gather(0)
        for c in range(n_ch):
            s = c % 2
            r0, r1, _, so = bufs[s]
            ga, gb = gat[s]
            ga.wait()
            gb.wait()
            if c + 1 < n_ch:
                if out[(c + 1) % 2] is not None:
                    out[(c + 1) % 2].wait()
                issue_gather(c + 1)

            def row_body(r, carry, r0=r0, r1=r1):
                for j in range(C // 16):
                    sl = pl.ds(j * 16, 16)
                    r0[r, sl] = (r0[r, sl] + r1[r, sl]) * 0.5
                return carry

            lax.fori_loop(0, ch, row_body, 0)
            out[s] = pltpu.async_copy(r0, out_hbm.at[pl.ds(base + c * ch, ch)], so)
        out[(n_ch - 1) % 2].wait()
        out[n_ch % 2].wait()

    return body(preds_flat, i0, i1)


def kernel(points, preds, k_vector):
    del k_vector
    pts_t = jnp.transpose(points, (0, 2, 1))
    i0, i1 = _top2_indices(points, pts_t)
    out = _gather_mean(preds.reshape(BN, C), i0.reshape(BN), i1.reshape(BN))
    return out.reshape(B, N, C)
